# Initial kernel scaffold; baseline (speedup 1.0000x reference)
import jax
import jax.numpy as jnp
from jax.experimental import pallas as pl


def kernel(x, embeddings, feature_emb):
    raise NotImplementedError("write your pallas kernel here")


# speedup vs baseline: 4.0511x; 4.0511x over previous
import functools

import jax
import jax.numpy as jnp
from jax import lax
from jax.experimental import pallas as pl
from jax.experimental.pallas import tpu as pltpu
from jax.experimental.pallas import tpu_sc as plsc

BATCH = 16384
N_FEATURES = 100
D_MODEL = 32

NUM_CORES = 2
NUM_SUBCORES = 16
NUM_WORKERS = NUM_CORES * NUM_SUBCORES

B_PER_WORKER = BATCH // NUM_WORKERS
B_PER_CHUNK = 8
NUM_CHUNKS = B_PER_WORKER // B_PER_CHUNK
HALF = D_MODEL // 2


def _tokenizer_body(x_hbm, emb_hbm, fe_hbm, out_hbm, idx_v, rows_v, bias_v, sem):
    wid = lax.axis_index("s") * NUM_CORES + lax.axis_index("c")
    b_base = wid * B_PER_WORKER

    pltpu.sync_copy(fe_hbm, bias_v)

    def chunk_body(g, carry):
        cb = b_base + g * B_PER_CHUNK
        pltpu.sync_copy(x_hbm.at[pl.ds(cb, B_PER_CHUNK)], idx_v)
        copies = [
            pltpu.make_async_copy(emb_hbm.at[idx_v.at[i]], rows_v.at[i], sem)
            for i in range(B_PER_CHUNK)
        ]
        for c in copies:
            c.start()
        for c in copies:
            c.wait()

        def f_body(f, carry2):
            b0 = bias_v[f, pl.ds(0, HALF)]
            b1 = bias_v[f, pl.ds(HALF, HALF)]
            for i in range(B_PER_CHUNK):
                plsc.addupdate(rows_v.at[i, f, pl.ds(0, HALF)], b0)
                plsc.addupdate(rows_v.at[i, f, pl.ds(HALF, HALF)], b1)
            return carry2

        lax.fori_loop(0, N_FEATURES, f_body, 0, unroll=False)

        pltpu.sync_copy(rows_v, out_hbm.at[pl.ds(cb, B_PER_CHUNK)])
        return carry

    lax.fori_loop(0, NUM_CHUNKS, chunk_body, 0, unroll=False)


@jax.jit
def _tokenizer(x, embeddings, feature_emb):
    mesh = plsc.VectorSubcoreMesh(
        core_axis_name="c", subcore_axis_name="s",
        num_cores=NUM_CORES, num_subcores=NUM_SUBCORES,
    )
    return pl.kernel(
        _tokenizer_body,
        out_type=jax.ShapeDtypeStruct((BATCH, N_FEATURES, D_MODEL), jnp.float32),
        mesh=mesh,
        compiler_params=pltpu.CompilerParams(use_tc_tiling_on_sc=False),
        scratch_types=[
            pltpu.VMEM((B_PER_CHUNK, N_FEATURES), jnp.int32),
            pltpu.VMEM((B_PER_CHUNK, N_FEATURES, D_MODEL), jnp.float32),
            pltpu.VMEM((N_FEATURES, D_MODEL), jnp.float32),
            pltpu.SemaphoreType.DMA,
        ],
    )(x, embeddings, feature_emb)


def kernel(x, embeddings, feature_emb):
    return _tokenizer(jnp.asarray(x, jnp.int32), embeddings, feature_emb)
